# baseline (speedup 1.0000x reference)
import functools

import jax
import jax.numpy as jnp
from jax import lax
from jax.experimental import pallas as pl
from jax.experimental.pallas import tpu as pltpu
from jax.experimental.pallas import tpu_sc as plsc

NUM_EXPERTS = 64
TOPK = 8
ROUTER_SCALE = 2.5
TOKENS_TOTAL = 32768
BT = 1024

_info = plsc.get_sparse_core_info()
_NC, _NS = _info.num_cores, _info.num_subcores
NW = _NC * _NS
TPW = TOKENS_TOTAL // NW
SLAB = TPW // 2


def _scores_block(x_ref, wt_ref, ow_ref):
    z = jnp.dot(x_ref[...], wt_ref[...],
                preferred_element_type=jnp.float32,
                precision=jax.lax.Precision.DEFAULT)
    z = z - jnp.max(z, axis=-1, keepdims=True)
    e = jnp.exp(z)
    ow_ref[...] = e / jnp.sum(e, axis=-1, keepdims=True)


def _tc_scores(x, wt):
    tokens, hidden = x.shape
    return pl.pallas_call(
        _scores_block,
        grid=(tokens // BT,),
        in_specs=[
            pl.BlockSpec((BT, hidden), lambda i: (i, 0)),
            pl.BlockSpec((hidden, NUM_EXPERTS), lambda i: (0, 0)),
        ],
        out_specs=pl.BlockSpec((BT, NUM_EXPERTS), lambda i: (i, 0)),
        out_shape=jax.ShapeDtypeStruct((tokens, NUM_EXPERTS), jnp.float32),
        compiler_params=pltpu.CompilerParams(
            dimension_semantics=("arbitrary",),
        ),
    )(x, wt)


def _sc_topk_body(scores_hbm, b_hbm, ow_hbm, oi_hbm,
                  sc_s, sc_b, sc_w, sc_i):
    wid = lax.axis_index("s") * _NC + lax.axis_index("c")
    base = wid * TPW
    pltpu.sync_copy(b_hbm, sc_b)

    iota = lax.iota(jnp.int32, 16)
    lane_lt8 = iota < 8
    bvs = [sc_b[pl.ds(c * 16, 16)] for c in range(4)]
    idxs = [iota + jnp.int32(c * 16) for c in range(4)]

    def merge(ka, pa, kb, pb):
        kb_r = lax.rev(kb, (0,))
        pb_r = lax.rev(pb, (0,))
        k = jnp.where(lane_lt8, ka, kb_r)
        p = jnp.where(lane_lt8, pa, pb_r)
        return lax.sort((k, p), dimension=0, num_keys=1)

    for h in range(TPW // SLAB):
        pltpu.sync_copy(scores_hbm.at[pl.ds(base + h * SLAB, SLAB), :], sc_s)

        @plsc.parallel_loop(0, SLAB, 1, unroll=8)
        def _body(t, h=h):
            chunks = []
            for c in range(4):
                v = sc_s[t, pl.ds(c * 16, 16)] + bvs[c]
                chunks.append(lax.sort((jnp.negative(v), idxs[c]),
                                       dimension=0, num_keys=1))
            (k0, p0), (k1, p1), (k2, p2), (k3, p3) = chunks
            ka, pa = merge(k0, p0, k1, p1)
            kb, pb = merge(k2, p2, k3, p3)
            _, pf = merge(ka, pa, kb, pb)
            tvec = jnp.full((16,), t, dtype=jnp.int32)
            w = plsc.load_gather(sc_s, [tvec, pf], mask=lane_lt8)
            w = w * ROUTER_SCALE
            out = (h * SLAB + t) * TOPK
            plsc.store_compressed(sc_w.at[pl.ds(out, 16)], w, mask=lane_lt8)
            plsc.store_compressed(sc_i.at[pl.ds(out, 16)], pf, mask=lane_lt8)

    pltpu.sync_copy(sc_w.at[pl.ds(0, TPW * TOPK)],
                    ow_hbm.at[pl.ds(base * TOPK, TPW * TOPK)])
    pltpu.sync_copy(sc_i.at[pl.ds(0, TPW * TOPK)],
                    oi_hbm.at[pl.ds(base * TOPK, TPW * TOPK)])


_sc_topk = functools.partial(
    pl.kernel,
    out_type=(
        jax.ShapeDtypeStruct((TOKENS_TOTAL * TOPK,), jnp.float32),
        jax.ShapeDtypeStruct((TOKENS_TOTAL * TOPK,), jnp.int32),
    ),
    mesh=plsc.VectorSubcoreMesh(core_axis_name="c", subcore_axis_name="s"),
    compiler_params=pltpu.CompilerParams(needs_layout_passes=False),
    scratch_types=[
        pltpu.VMEM((SLAB, NUM_EXPERTS), jnp.float32),
        pltpu.VMEM((NUM_EXPERTS,), jnp.float32),
        pltpu.VMEM((TPW * TOPK + TOPK,), jnp.float32),
        pltpu.VMEM((TPW * TOPK + TOPK,), jnp.int32),
    ],
)(_sc_topk_body)


@jax.jit
def kernel(x, W, b):
    tokens, _ = x.shape
    scores = _tc_scores(x, W.T)
    ow, oi = _sc_topk(scores, b)
    return (ow.reshape(tokens, TOPK), oi.reshape(tokens, TOPK))


# speedup vs baseline: 1.0011x; 1.0011x over previous
import functools

import jax
import jax.numpy as jnp
from jax import lax
from jax.experimental import pallas as pl
from jax.experimental.pallas import tpu as pltpu
from jax.experimental.pallas import tpu_sc as plsc

NUM_EXPERTS = 64
TOPK = 8
ROUTER_SCALE = 2.5
TOKENS_TOTAL = 32768
BT = 1024
NCHUNK = 4
CT = TOKENS_TOTAL // NCHUNK

_info = plsc.get_sparse_core_info()
_NC, _NS = _info.num_cores, _info.num_subcores
NW = _NC * _NS
TPW = CT // NW


def _scores_block(x_ref, wt_ref, ow_ref):
    z = jnp.dot(x_ref[...], wt_ref[...],
                preferred_element_type=jnp.float32,
                precision=jax.lax.Precision.DEFAULT)
    z = z - jnp.max(z, axis=-1, keepdims=True)
    e = jnp.exp(z)
    ow_ref[...] = e / jnp.sum(e, axis=-1, keepdims=True)


def _tc_scores(x, wt, chunk):
    hidden = x.shape[1]
    blk0 = chunk * (CT // BT)
    return pl.pallas_call(
        _scores_block,
        grid=(CT // BT,),
        in_specs=[
            pl.BlockSpec((BT, hidden), lambda i: (blk0 + i, 0)),
            pl.BlockSpec((hidden, NUM_EXPERTS), lambda i: (0, 0)),
        ],
        out_specs=pl.BlockSpec((BT, NUM_EXPERTS), lambda i: (i, 0)),
        out_shape=jax.ShapeDtypeStruct((CT, NUM_EXPERTS), jnp.float32),
        compiler_params=pltpu.CompilerParams(
            dimension_semantics=("arbitrary",),
        ),
    )(x, wt)


def _sc_topk_body(scores_hbm, b_hbm, ow_hbm, oi_hbm,
                  sc_s, sc_b, sc_w, sc_i):
    wid = lax.axis_index("s") * _NC + lax.axis_index("c")
    base = wid * TPW
    pltpu.sync_copy(scores_hbm.at[pl.ds(base, TPW), :], sc_s)
    pltpu.sync_copy(b_hbm, sc_b)

    iota = lax.iota(jnp.int32, 16)
    lane_lt8 = iota < 8
    nbvs = [jnp.negative(sc_b[pl.ds(c * 16, 16)]) for c in range(4)]
    idxs = [iota + jnp.int32(c * 16) for c in range(4)]

    def merge(ka, pa, kb, pb):
        kb_r = lax.rev(kb, (0,))
        pb_r = lax.rev(pb, (0,))
        k = jnp.where(lane_lt8, ka, kb_r)
        p = jnp.where(lane_lt8, pa, pb_r)
        return lax.sort((k, p), dimension=0, num_keys=1)

    @plsc.parallel_loop(0, TPW, 1, unroll=8)
    def _body(t):
        chunks = []
        for c in range(4):
            nv = nbvs[c] - sc_s[t, pl.ds(c * 16, 16)]
            chunks.append(lax.sort((nv, idxs[c]), dimension=0, num_keys=1))
        (k0, p0), (k1, p1), (k2, p2), (k3, p3) = chunks
        ka, pa = merge(k0, p0, k1, p1)
        kb, pb = merge(k2, p2, k3, p3)
        _, pf = merge(ka, pa, kb, pb)
        tvec = jnp.full((16,), t, dtype=jnp.int32)
        w = plsc.load_gather(sc_s, [tvec, pf], mask=lane_lt8)
        w = w * ROUTER_SCALE
        plsc.store_compressed(sc_w.at[pl.ds(t * TOPK, 16)], w, mask=lane_lt8)
        plsc.store_compressed(sc_i.at[pl.ds(t * TOPK, 16)], pf, mask=lane_lt8)

    pltpu.sync_copy(sc_w.at[pl.ds(0, TPW * TOPK)],
                    ow_hbm.at[pl.ds(base * TOPK, TPW * TOPK)])
    pltpu.sync_copy(sc_i.at[pl.ds(0, TPW * TOPK)],
                    oi_hbm.at[pl.ds(base * TOPK, TPW * TOPK)])


_sc_topk = functools.partial(
    pl.kernel,
    out_type=(
        jax.ShapeDtypeStruct((CT * TOPK,), jnp.float32),
        jax.ShapeDtypeStruct((CT * TOPK,), jnp.int32),
    ),
    mesh=plsc.VectorSubcoreMesh(core_axis_name="c", subcore_axis_name="s"),
    compiler_params=pltpu.CompilerParams(needs_layout_passes=False),
    scratch_types=[
        pltpu.VMEM((TPW, NUM_EXPERTS), jnp.float32),
        pltpu.VMEM((NUM_EXPERTS,), jnp.float32),
        pltpu.VMEM((TPW * TOPK + TOPK,), jnp.float32),
        pltpu.VMEM((TPW * TOPK + TOPK,), jnp.int32),
    ],
)(_sc_topk_body)


@jax.jit
def kernel(x, W, b):
    wt = W.T
    w_chunks = []
    i_chunks = []
    for c in range(NCHUNK):
        scores = _tc_scores(x, wt, c)
        ow, oi = _sc_topk(scores, b)
        w_chunks.append(ow.reshape(CT, TOPK))
        i_chunks.append(oi.reshape(CT, TOPK))
    return (jnp.concatenate(w_chunks, axis=0),
            jnp.concatenate(i_chunks, axis=0))
